# baseline (speedup 1.0000x reference)
import jax
import jax.numpy as jnp
from jax.experimental import pallas as pl
from jax.experimental.pallas import tpu as pltpu

_CHUNK = 7680
_W = 512


def _chunks(rows):
    out = []
    off = 0
    while off < rows:
        sz = min(_CHUNK, rows - off)
        out.append((off, sz))
        off += sz
    return out


def _body_factory(rows):
    chunks = _chunks(rows)
    n = len(chunks)

    def body(sites_ref, x_ref, o_ref, in_buf, out_buf, in_sems, out_sems):
        n_sites = sites_ref.shape[0]
        col = jax.lax.broadcasted_iota(jnp.int32, (8, _W), 1)

        def mbody(i, m):
            return jnp.where(col == sites_ref[i], 0.0, m)

        mask = jax.lax.fori_loop(
            0, n_sites, mbody, jnp.ones((8, _W), jnp.float32)
        )[0:1, :]

        def in_copy(i):
            off, sz = chunks[i]
            return pltpu.make_async_copy(
                x_ref.at[pl.ds(off, sz)],
                in_buf.at[i % 2, pl.ds(0, sz)],
                in_sems.at[i % 2],
            )

        def out_copy(i):
            off, sz = chunks[i]
            return pltpu.make_async_copy(
                out_buf.at[i % 2, pl.ds(0, sz)],
                o_ref.at[pl.ds(off, sz)],
                out_sems.at[i % 2],
            )

        in_copy(0).start()
        for i in range(n):
            if i + 1 < n:
                in_copy(i + 1).start()
            in_copy(i).wait()
            if i >= 2:
                out_copy(i - 2).wait()
            _, sz = chunks[i]
            out_buf[i % 2, pl.ds(0, sz)] = in_buf[i % 2, pl.ds(0, sz)] * mask
            out_copy(i).start()
        out_copy(n - 2).wait()
        out_copy(n - 1).wait()

    return body


def kernel(x, mask_sites):
    C, H, W = x.shape
    rows = C * H
    x2 = x.reshape(rows, W)
    out = pl.pallas_call(
        _body_factory(rows),
        in_specs=[
            pl.BlockSpec(memory_space=pltpu.SMEM),
            pl.BlockSpec(memory_space=pl.ANY),
        ],
        out_specs=pl.BlockSpec(memory_space=pl.ANY),
        out_shape=jax.ShapeDtypeStruct((rows, W), x.dtype),
        scratch_shapes=[
            pltpu.VMEM((2, _CHUNK, W), jnp.float32),
            pltpu.VMEM((2, _CHUNK, W), jnp.float32),
            pltpu.SemaphoreType.DMA((2,)),
            pltpu.SemaphoreType.DMA((2,)),
        ],
        compiler_params=pltpu.CompilerParams(
            vmem_limit_bytes=64 * 1024 * 1024
        ),
    )(mask_sites, x2)
    return out.reshape(C, H, W)


# speedup vs baseline: 1.0088x; 1.0088x over previous
import jax
import jax.numpy as jnp
from jax.experimental import pallas as pl
from jax.experimental.pallas import tpu as pltpu

_ROWS_PER_BLOCK = 6144


def _mask_mul_body(sites_ref, x_ref, o_ref, mask_ref):
    n_sites = sites_ref.shape[0]
    w = mask_ref.shape[1]

    @pl.when(pl.program_id(0) == 0)
    def _build_mask():
        col = jax.lax.broadcasted_iota(jnp.int32, (8, w), 1)

        def body(i, m):
            return jnp.where(col == sites_ref[i], 0.0, m)

        mask_ref[...] = jax.lax.fori_loop(
            0, n_sites, body, jnp.ones((8, w), jnp.float32)
        )

    o_ref[...] = x_ref[...] * mask_ref[0:1, :]


def kernel(x, mask_sites):
    C, H, W = x.shape
    rows = C * H
    x2 = x.reshape(rows, W)
    n_blocks = rows // _ROWS_PER_BLOCK
    out = pl.pallas_call(
        _mask_mul_body,
        grid=(n_blocks,),
        in_specs=[
            pl.BlockSpec(memory_space=pltpu.SMEM),
            pl.BlockSpec((_ROWS_PER_BLOCK, W), lambda i: (i, 0)),
        ],
        out_specs=pl.BlockSpec((_ROWS_PER_BLOCK, W), lambda i: (i, 0)),
        out_shape=jax.ShapeDtypeStruct((rows, W), x.dtype),
        scratch_shapes=[pltpu.VMEM((8, W), jnp.float32)],
        compiler_params=pltpu.CompilerParams(
            vmem_limit_bytes=128 * 1024 * 1024
        ),
    )(mask_sites, x2)
    return out.reshape(C, H, W)
